# Initial kernel scaffold; baseline (speedup 1.0000x reference)
import jax
import jax.numpy as jnp
from jax.experimental import pallas as pl


def kernel(x, edge_index, batch, W1, b1, W2, b2, Wf1, bf1, Wf2, bf2):
    raise NotImplementedError("write your pallas kernel here")


# speedup vs baseline: 13.3611x; 13.3611x over previous
import functools

import jax
import jax.numpy as jnp
from jax import lax
from jax.experimental import pallas as pl
from jax.experimental.pallas import tpu as pltpu
from jax.experimental.pallas import tpu_sc as plsc

NC, NS, L = 2, 16, 16
NW = NC * NS

N, E, D, H, G = 10000, 160000, 256, 256, 64
HD = D // 2

EPW = E // NW
EPW_P = 5008
EPS = E // NS
BSZ = 128
NB = 79
EPS_P = NB * BSZ
PADE = EPS_P - EPS
NP_P = 10112
RPS = NP_P // NS
NPOOL = 10240
BPW = NPOOL // NW
PPS = NPOOL // NS
PB = PPS // BSZ
GP = 80

_mesh = plsc.VectorSubcoreMesh(core_axis_name="c", subcore_axis_name="s",
                               num_cores=NC, num_subcores=NS)
_sc_params = pltpu.CompilerParams(needs_layout_passes=False)


def _zero_vmem(ref, n):
    def body(i, _):
        ref[pl.ds(i * L, L)] = jnp.zeros((L,), jnp.float32)
        return 0
    lax.fori_loop(0, n // L, body, 0)


@functools.partial(
    pl.kernel,
    out_type=[jax.ShapeDtypeStruct((NW * NP_P,), jnp.float32),
              jax.ShapeDtypeStruct((NW * GP,), jnp.float32)],
    mesh=_mesh,
    compiler_params=_sc_params,
    scratch_types=[pltpu.VMEM((EPW_P,), jnp.int32),
                   pltpu.VMEM((BPW,), jnp.int32),
                   pltpu.VMEM((NP_P,), jnp.float32),
                   pltpu.VMEM((GP,), jnp.float32)],
)
def _sc_pre(dst_hbm, bat_hbm, deg_out, cnt_out, dstv, batv, dacc, cacc):
    c = lax.axis_index("c")
    s = lax.axis_index("s")
    wid = s * NC + c
    pltpu.sync_copy(dst_hbm.at[pl.ds(wid * EPW_P, EPW_P)], dstv)
    pltpu.sync_copy(bat_hbm.at[pl.ds(wid * BPW, BPW)], batv)
    _zero_vmem(dacc, NP_P)
    _zero_vmem(cacc, GP)
    ones = jnp.ones((L,), jnp.float32)

    def deg_body(i, _):
        d = dstv[pl.ds(i * L, L)]
        idx = jnp.where(d >= 0, d, N)
        plsc.addupdate_scatter(dacc, [idx], ones)
        return 0
    lax.fori_loop(0, EPW_P // L, deg_body, 0)

    def cnt_body(i, _):
        b = batv[pl.ds(i * L, L)]
        plsc.addupdate_scatter(cacc, [b], ones)
        return 0
    lax.fori_loop(0, BPW // L, cnt_body, 0)

    pltpu.sync_copy(dacc, deg_out.at[pl.ds(wid * NP_P, NP_P)])
    pltpu.sync_copy(cacc, cnt_out.at[pl.ds(wid * GP, GP)])


@functools.partial(
    pl.kernel,
    out_type=jax.ShapeDtypeStruct((NC, NP_P, HD), jnp.float32),
    mesh=_mesh,
    compiler_params=_sc_params,
    scratch_types=[pltpu.VMEM((NB, BSZ), jnp.int32),
                   pltpu.VMEM((NB, BSZ), jnp.int32),
                   pltpu.VMEM((BSZ, HD), jnp.float32),
                   pltpu.VMEM_SHARED((NP_P, HD), jnp.float32),
                   pltpu.SemaphoreType.DMA],
)
def _sc_agg(glo_hbm, ghi_hbm, src_hbm, dst_hbm, zeros_hbm, out_hbm,
            srcv, dstv, rbuf, acc, sem):
    c = lax.axis_index("c")
    s = lax.axis_index("s")
    pltpu.sync_copy(zeros_hbm.at[pl.ds(0, RPS)], acc.at[pl.ds(s * RPS, RPS)])
    pltpu.sync_copy(src_hbm.at[s], srcv)
    pltpu.sync_copy(dst_hbm.at[s], dstv)
    plsc.subcore_barrier()

    def run(gref):
        def body(b, _):
            pltpu.async_copy(gref.at[srcv.at[b]], rbuf, sem).wait()
            pltpu.sync_copy(rbuf, acc.at[dstv.at[b]], add=True)
            return 0
        lax.fori_loop(0, NB, body, 0)

    @pl.when(c == 0)
    def _():
        run(glo_hbm)

    @pl.when(c == 1)
    def _():
        run(ghi_hbm)

    plsc.subcore_barrier()
    pltpu.sync_copy(acc.at[pl.ds(s * RPS, RPS)],
                    out_hbm.at[c, pl.ds(s * RPS, RPS)])


@functools.partial(
    pl.kernel,
    out_type=jax.ShapeDtypeStruct((NC, G, HD), jnp.float32),
    mesh=_mesh,
    compiler_params=_sc_params,
    scratch_types=[pltpu.VMEM((PB, BSZ), jnp.int32),
                   pltpu.VMEM((PPS, HD), jnp.float32),
                   pltpu.VMEM_SHARED((GP, HD), jnp.float32)],
)
def _sc_pool(clo_hbm, chi_hbm, bidx_hbm, zeros_hbm, out_hbm,
             bidxv, rbuf, pacc):
    c = lax.axis_index("c")
    s = lax.axis_index("s")

    @pl.when(s == 0)
    def _():
        pltpu.sync_copy(zeros_hbm.at[pl.ds(0, GP)], pacc)

    @pl.when(c == 0)
    def _():
        pltpu.sync_copy(clo_hbm.at[pl.ds(s * PPS, PPS)], rbuf)

    @pl.when(c == 1)
    def _():
        pltpu.sync_copy(chi_hbm.at[pl.ds(s * PPS, PPS)], rbuf)

    pltpu.sync_copy(bidx_hbm.at[s], bidxv)
    plsc.subcore_barrier()
    for t in range(PB):
        pltpu.sync_copy(rbuf.at[pl.ds(t * BSZ, BSZ)],
                        pacc.at[bidxv.at[t]], add=True)
    plsc.subcore_barrier()

    @pl.when(s == 0)
    def _():
        pltpu.sync_copy(pacc.at[pl.ds(0, G)], out_hbm.at[c])


R = 1000


def _tc_a_body(degp, x, w1, glo, ghi, dinv):
    deg = 1.0 + jnp.sum(degp[...], axis=1)
    di = lax.rsqrt(deg)[:, None]
    z = jnp.dot(x[...], w1[...], preferred_element_type=jnp.float32)
    g = z * di
    glo[...] = g[:, :HD]
    ghi[...] = g[:, HD:]
    dinv[...] = di


def _tc_a(deg_t, x, w1):
    return pl.pallas_call(
        _tc_a_body,
        grid=(N // R,),
        in_specs=[pl.BlockSpec((R, NW), lambda i: (i, 0)),
                  pl.BlockSpec((R, D), lambda i: (i, 0)),
                  pl.BlockSpec((D, H), lambda i: (0, 0))],
        out_specs=[pl.BlockSpec((R, HD), lambda i: (i, 0)),
                   pl.BlockSpec((R, HD), lambda i: (i, 0)),
                   pl.BlockSpec((R, 1), lambda i: (i, 0))],
        out_shape=[jax.ShapeDtypeStruct((N, HD), jnp.float32),
                   jax.ShapeDtypeStruct((N, HD), jnp.float32),
                   jax.ShapeDtypeStruct((N, 1), jnp.float32)],
    )(deg_t, x, w1)


def _tc_b_body(alo, ahi, glo, ghi, dinv, b1, w2, g2lo, g2hi):
    agg = jnp.concatenate([alo[0], ahi[0]], axis=1)
    g = jnp.concatenate([glo[...], ghi[...]], axis=1)
    h1 = jax.nn.relu(dinv[...] * (agg + g) + b1[...])
    z2 = jnp.dot(h1, w2[...], preferred_element_type=jnp.float32)
    g2 = z2 * dinv[...]
    g2lo[...] = g2[:, :HD]
    g2hi[...] = g2[:, HD:]


def _tc_b(a1, glo, ghi, dinv, b1, w2):
    return pl.pallas_call(
        _tc_b_body,
        grid=(N // R,),
        in_specs=[pl.BlockSpec((1, R, HD), lambda i: (0, i, 0)),
                  pl.BlockSpec((1, R, HD), lambda i: (1, i, 0)),
                  pl.BlockSpec((R, HD), lambda i: (i, 0)),
                  pl.BlockSpec((R, HD), lambda i: (i, 0)),
                  pl.BlockSpec((R, 1), lambda i: (i, 0)),
                  pl.BlockSpec((1, H), lambda i: (0, 0)),
                  pl.BlockSpec((H, H), lambda i: (0, 0))],
        out_specs=[pl.BlockSpec((R, HD), lambda i: (i, 0)),
                   pl.BlockSpec((R, HD), lambda i: (i, 0))],
        out_shape=[jax.ShapeDtypeStruct((N, HD), jnp.float32),
                   jax.ShapeDtypeStruct((N, HD), jnp.float32)],
    )(a1, a1, glo, ghi, dinv, b1, w2)


def _tc_c_body(alo, ahi, glo, ghi, dinv, b2, clo, chi):
    agg = jnp.concatenate([alo[0], ahi[0]], axis=1)
    g = jnp.concatenate([glo[...], ghi[...]], axis=1)
    conv = dinv[...] * (agg + g) + b2[...]
    clo[...] = conv[:, :HD]
    chi[...] = conv[:, HD:]


def _tc_c(a2, g2lo, g2hi, dinv, b2):
    return pl.pallas_call(
        _tc_c_body,
        grid=(N // R,),
        in_specs=[pl.BlockSpec((1, R, HD), lambda i: (0, i, 0)),
                  pl.BlockSpec((1, R, HD), lambda i: (1, i, 0)),
                  pl.BlockSpec((R, HD), lambda i: (i, 0)),
                  pl.BlockSpec((R, HD), lambda i: (i, 0)),
                  pl.BlockSpec((R, 1), lambda i: (i, 0)),
                  pl.BlockSpec((1, H), lambda i: (0, 0))],
        out_specs=[pl.BlockSpec((R, HD), lambda i: (i, 0)),
                   pl.BlockSpec((R, HD), lambda i: (i, 0))],
        out_shape=[jax.ShapeDtypeStruct((NPOOL, HD), jnp.float32),
                   jax.ShapeDtypeStruct((NPOOL, HD), jnp.float32)],
    )(a2, a2, g2lo, g2hi, dinv, b2)


def _tc_d_body(pp, cntp, wf1, bf1, wf2, bf2, out):
    p = pp[...]
    cnt = jnp.sum(cntp[...], axis=1)
    pooled = jnp.concatenate([p[0], p[1]], axis=1)
    pooled = pooled / jnp.maximum(cnt, 1.0)[:, None]
    s1 = jax.nn.relu(
        jnp.dot(pooled, wf1[...], preferred_element_type=jnp.float32)
        + bf1[...])
    out[...] = jnp.dot(s1, wf2[...],
                       preferred_element_type=jnp.float32) + bf2[...]


def _tc_d(pool_p, cnt_t, wf1, bf1, wf2, bf2):
    return pl.pallas_call(
        _tc_d_body,
        out_shape=jax.ShapeDtypeStruct((G, 1), jnp.float32),
    )(pool_p, cnt_t, wf1, bf1, wf2, bf2)


@jax.jit
def kernel(x, edge_index, batch, W1, b1, W2, b2, Wf1, bf1, Wf2, bf2):
    src, dst = edge_index[0], edge_index[1]

    spread = jnp.arange(PADE, dtype=jnp.int32) % 16
    pad_blk = jnp.broadcast_to(spread, (NS, PADE))
    src_s = jnp.concatenate([src.reshape(NS, EPS), pad_blk],
                            axis=1).reshape(NS, NB, BSZ)
    dst_s = jnp.concatenate([dst.reshape(NS, EPS), N + pad_blk],
                            axis=1).reshape(NS, NB, BSZ)
    dst_deg = jnp.pad(dst.reshape(NW, EPW), ((0, 0), (0, EPW_P - EPW)),
                      constant_values=-1).reshape(-1)
    pad_bat = G + (jnp.arange(NPOOL - N, dtype=jnp.int32) % 16)
    bat_all = jnp.concatenate([batch, pad_bat])
    bidx = bat_all.reshape(NS, PB, BSZ)
    zeros = jnp.zeros((RPS, HD), jnp.float32)

    deg_p, cnt_p = _sc_pre(dst_deg, bat_all)
    deg_t = deg_p.reshape(NW, NP_P)[:, :N].T
    cnt_t = cnt_p.reshape(NW, GP)[:, :G].T

    g1lo, g1hi, dinv = _tc_a(deg_t, x, W1)
    a1 = _sc_agg(g1lo, g1hi, src_s, dst_s, zeros)
    g2lo, g2hi = _tc_b(a1, g1lo, g1hi, dinv, b1.reshape(1, H), W2)
    a2 = _sc_agg(g2lo, g2hi, src_s, dst_s, zeros)
    c2lo, c2hi = _tc_c(a2, g2lo, g2hi, dinv, b2.reshape(1, H))
    pool_p = _sc_pool(c2lo, c2hi, bidx, zeros)
    out = _tc_d(pool_p, cnt_t, Wf1, bf1.reshape(1, 128), Wf2,
                bf2.reshape(1, 1))
    return out


# speedup vs baseline: 18.2011x; 1.3623x over previous
import functools

import jax
import jax.numpy as jnp
from jax import lax
from jax.experimental import pallas as pl
from jax.experimental.pallas import tpu as pltpu
from jax.experimental.pallas import tpu_sc as plsc

NC, NS, L = 2, 16, 16
NW = NC * NS

N, E, D, H, G = 10000, 160000, 256, 256, 64
HD = D // 2

EPW = E // NW
EPW_P = 5008
EPS = E // NS
BSZ = 128
NB = 80
NBH = NB // 2
EPS_P = NB * BSZ
PADE = EPS_P - EPS
NP_P = 10112
RPS = NP_P // NS
NPOOL = 10240
BPW = NPOOL // NW
PPS = NPOOL // NS
PB = PPS // BSZ
GP = 80

_mesh = plsc.VectorSubcoreMesh(core_axis_name="c", subcore_axis_name="s",
                               num_cores=NC, num_subcores=NS)
_sc_params = pltpu.CompilerParams(needs_layout_passes=False)


def _zero_vmem(ref, n):
    def body(i, _):
        ref[pl.ds(i * L, L)] = jnp.zeros((L,), jnp.float32)
        return 0
    lax.fori_loop(0, n // L, body, 0)


@functools.partial(
    pl.kernel,
    out_type=[jax.ShapeDtypeStruct((NW * NP_P,), jnp.float32),
              jax.ShapeDtypeStruct((NW * GP,), jnp.float32)],
    mesh=_mesh,
    compiler_params=_sc_params,
    scratch_types=[pltpu.VMEM((EPW_P,), jnp.int32),
                   pltpu.VMEM((BPW,), jnp.int32),
                   pltpu.VMEM((NP_P,), jnp.float32),
                   pltpu.VMEM((GP,), jnp.float32)],
)
def _sc_pre(dst_hbm, bat_hbm, deg_out, cnt_out, dstv, batv, dacc, cacc):
    c = lax.axis_index("c")
    s = lax.axis_index("s")
    wid = s * NC + c
    pltpu.sync_copy(dst_hbm.at[pl.ds(wid * EPW_P, EPW_P)], dstv)
    pltpu.sync_copy(bat_hbm.at[pl.ds(wid * BPW, BPW)], batv)
    _zero_vmem(dacc, NP_P)
    _zero_vmem(cacc, GP)
    ones = jnp.ones((L,), jnp.float32)

    def deg_body(i, _):
        d = dstv[pl.ds(i * L, L)]
        idx = jnp.where(d >= 0, d, N)
        plsc.addupdate_scatter(dacc, [idx], ones)
        return 0
    lax.fori_loop(0, EPW_P // L, deg_body, 0)

    def cnt_body(i, _):
        b = batv[pl.ds(i * L, L)]
        plsc.addupdate_scatter(cacc, [b], ones)
        return 0
    lax.fori_loop(0, BPW // L, cnt_body, 0)

    pltpu.sync_copy(dacc, deg_out.at[pl.ds(wid * NP_P, NP_P)])
    pltpu.sync_copy(cacc, cnt_out.at[pl.ds(wid * GP, GP)])


@functools.partial(
    pl.kernel,
    out_type=jax.ShapeDtypeStruct((NC, NP_P, HD), jnp.float32),
    mesh=_mesh,
    compiler_params=_sc_params,
    scratch_types=[pltpu.VMEM((NBH, BSZ), jnp.int32),
                   pltpu.VMEM((NBH, BSZ), jnp.int32),
                   pltpu.VMEM((BSZ, HD), jnp.float32),
                   pltpu.VMEM((BSZ, HD), jnp.float32),
                   pltpu.VMEM_SHARED((NP_P, HD), jnp.float32),
                   pltpu.SemaphoreType.DMA,
                   pltpu.SemaphoreType.DMA],
)
def _sc_agg(glo_hbm, ghi_hbm, src_hbm, dst_hbm, zeros_hbm, out_hbm,
            srcv, dstv, rbuf0, rbuf1, acc, sem0, sem1):
    c = lax.axis_index("c")
    s = lax.axis_index("s")
    pltpu.sync_copy(zeros_hbm.at[pl.ds(0, RPS)], acc.at[pl.ds(s * RPS, RPS)])
    plsc.subcore_barrier()

    def run(gref):
        def phase(p):
            pltpu.sync_copy(src_hbm.at[s, pl.ds(p * NBH, NBH)], srcv)
            pltpu.sync_copy(dst_hbm.at[s, pl.ds(p * NBH, NBH)], dstv)
            pltpu.async_copy(gref.at[srcv.at[0]], rbuf0, sem0)
            pltpu.async_copy(gref.at[srcv.at[1]], rbuf1, sem1)

            def step(b, rbuf, sem, refill):
                pltpu.make_async_copy(gref.at[srcv.at[b]], rbuf, sem).wait()
                pltpu.sync_copy(rbuf, acc.at[dstv.at[b]], add=True)
                if refill:
                    pltpu.async_copy(gref.at[srcv.at[b + 2]], rbuf, sem)

            def body(g, _):
                step(2 * g, rbuf0, sem0, True)
                step(2 * g + 1, rbuf1, sem1, True)
                return 0
            lax.fori_loop(0, (NBH - 2) // 2, body, 0)
            step(NBH - 2, rbuf0, sem0, False)
            step(NBH - 1, rbuf1, sem1, False)

        phase(0)
        phase(1)

    @pl.when(c == 0)
    def _():
        run(glo_hbm)

    @pl.when(c == 1)
    def _():
        run(ghi_hbm)

    plsc.subcore_barrier()
    pltpu.sync_copy(acc.at[pl.ds(s * RPS, RPS)],
                    out_hbm.at[c, pl.ds(s * RPS, RPS)])


@functools.partial(
    pl.kernel,
    out_type=jax.ShapeDtypeStruct((NC, G, HD), jnp.float32),
    mesh=_mesh,
    compiler_params=_sc_params,
    scratch_types=[pltpu.VMEM((PB, BSZ), jnp.int32),
                   pltpu.VMEM((PPS, HD), jnp.float32),
                   pltpu.VMEM_SHARED((GP, HD), jnp.float32)],
)
def _sc_pool(clo_hbm, chi_hbm, bidx_hbm, zeros_hbm, out_hbm,
             bidxv, rbuf, pacc):
    c = lax.axis_index("c")
    s = lax.axis_index("s")

    @pl.when(s == 0)
    def _():
        pltpu.sync_copy(zeros_hbm.at[pl.ds(0, GP)], pacc)

    @pl.when(c == 0)
    def _():
        pltpu.sync_copy(clo_hbm.at[pl.ds(s * PPS, PPS)], rbuf)

    @pl.when(c == 1)
    def _():
        pltpu.sync_copy(chi_hbm.at[pl.ds(s * PPS, PPS)], rbuf)

    pltpu.sync_copy(bidx_hbm.at[s], bidxv)
    plsc.subcore_barrier()
    for t in range(PB):
        pltpu.sync_copy(rbuf.at[pl.ds(t * BSZ, BSZ)],
                        pacc.at[bidxv.at[t]], add=True)
    plsc.subcore_barrier()

    @pl.when(s == 0)
    def _():
        pltpu.sync_copy(pacc.at[pl.ds(0, G)], out_hbm.at[c])


R = 1000


def _tc_a_body(degp, x, w1, glo, ghi, dinv):
    deg = 1.0 + jnp.sum(degp[...], axis=1)
    di = lax.rsqrt(deg)[:, None]
    z = jnp.dot(x[...], w1[...], preferred_element_type=jnp.float32)
    g = z * di
    glo[...] = g[:, :HD]
    ghi[...] = g[:, HD:]
    dinv[...] = di


def _tc_a(deg_t, x, w1):
    return pl.pallas_call(
        _tc_a_body,
        grid=(N // R,),
        in_specs=[pl.BlockSpec((R, NW), lambda i: (i, 0)),
                  pl.BlockSpec((R, D), lambda i: (i, 0)),
                  pl.BlockSpec((D, H), lambda i: (0, 0))],
        out_specs=[pl.BlockSpec((R, HD), lambda i: (i, 0)),
                   pl.BlockSpec((R, HD), lambda i: (i, 0)),
                   pl.BlockSpec((R, 1), lambda i: (i, 0))],
        out_shape=[jax.ShapeDtypeStruct((N, HD), jnp.float32),
                   jax.ShapeDtypeStruct((N, HD), jnp.float32),
                   jax.ShapeDtypeStruct((N, 1), jnp.float32)],
    )(deg_t, x, w1)


def _tc_b_body(alo, ahi, glo, ghi, dinv, b1, w2, g2lo, g2hi):
    agg = jnp.concatenate([alo[0], ahi[0]], axis=1)
    g = jnp.concatenate([glo[...], ghi[...]], axis=1)
    h1 = jax.nn.relu(dinv[...] * (agg + g) + b1[...])
    z2 = jnp.dot(h1, w2[...], preferred_element_type=jnp.float32)
    g2 = z2 * dinv[...]
    g2lo[...] = g2[:, :HD]
    g2hi[...] = g2[:, HD:]


def _tc_b(a1, glo, ghi, dinv, b1, w2):
    return pl.pallas_call(
        _tc_b_body,
        grid=(N // R,),
        in_specs=[pl.BlockSpec((1, R, HD), lambda i: (0, i, 0)),
                  pl.BlockSpec((1, R, HD), lambda i: (1, i, 0)),
                  pl.BlockSpec((R, HD), lambda i: (i, 0)),
                  pl.BlockSpec((R, HD), lambda i: (i, 0)),
                  pl.BlockSpec((R, 1), lambda i: (i, 0)),
                  pl.BlockSpec((1, H), lambda i: (0, 0)),
                  pl.BlockSpec((H, H), lambda i: (0, 0))],
        out_specs=[pl.BlockSpec((R, HD), lambda i: (i, 0)),
                   pl.BlockSpec((R, HD), lambda i: (i, 0))],
        out_shape=[jax.ShapeDtypeStruct((N, HD), jnp.float32),
                   jax.ShapeDtypeStruct((N, HD), jnp.float32)],
    )(a1, a1, glo, ghi, dinv, b1, w2)


def _tc_c_body(alo, ahi, glo, ghi, dinv, b2, clo, chi):
    agg = jnp.concatenate([alo[0], ahi[0]], axis=1)
    g = jnp.concatenate([glo[...], ghi[...]], axis=1)
    conv = dinv[...] * (agg + g) + b2[...]
    clo[...] = conv[:, :HD]
    chi[...] = conv[:, HD:]


def _tc_c(a2, g2lo, g2hi, dinv, b2):
    return pl.pallas_call(
        _tc_c_body,
        grid=(N // R,),
        in_specs=[pl.BlockSpec((1, R, HD), lambda i: (0, i, 0)),
                  pl.BlockSpec((1, R, HD), lambda i: (1, i, 0)),
                  pl.BlockSpec((R, HD), lambda i: (i, 0)),
                  pl.BlockSpec((R, HD), lambda i: (i, 0)),
                  pl.BlockSpec((R, 1), lambda i: (i, 0)),
                  pl.BlockSpec((1, H), lambda i: (0, 0))],
        out_specs=[pl.BlockSpec((R, HD), lambda i: (i, 0)),
                   pl.BlockSpec((R, HD), lambda i: (i, 0))],
        out_shape=[jax.ShapeDtypeStruct((NPOOL, HD), jnp.float32),
                   jax.ShapeDtypeStruct((NPOOL, HD), jnp.float32)],
    )(a2, a2, g2lo, g2hi, dinv, b2)


def _tc_d_body(pp, cntp, wf1, bf1, wf2, bf2, out):
    p = pp[...]
    cnt = jnp.sum(cntp[...], axis=1)
    pooled = jnp.concatenate([p[0], p[1]], axis=1)
    pooled = pooled / jnp.maximum(cnt, 1.0)[:, None]
    s1 = jax.nn.relu(
        jnp.dot(pooled, wf1[...], preferred_element_type=jnp.float32)
        + bf1[...])
    out[...] = jnp.dot(s1, wf2[...],
                       preferred_element_type=jnp.float32) + bf2[...]


def _tc_d(pool_p, cnt_t, wf1, bf1, wf2, bf2):
    return pl.pallas_call(
        _tc_d_body,
        out_shape=jax.ShapeDtypeStruct((G, 1), jnp.float32),
    )(pool_p, cnt_t, wf1, bf1, wf2, bf2)


@jax.jit
def kernel(x, edge_index, batch, W1, b1, W2, b2, Wf1, bf1, Wf2, bf2):
    src, dst = edge_index[0], edge_index[1]

    spread = jnp.arange(PADE, dtype=jnp.int32) % 16
    pad_blk = jnp.broadcast_to(spread, (NS, PADE))
    src_s = jnp.concatenate([src.reshape(NS, EPS), pad_blk],
                            axis=1).reshape(NS, NB, BSZ)
    dst_s = jnp.concatenate([dst.reshape(NS, EPS), N + pad_blk],
                            axis=1).reshape(NS, NB, BSZ)
    dst_deg = jnp.pad(dst.reshape(NW, EPW), ((0, 0), (0, EPW_P - EPW)),
                      constant_values=-1).reshape(-1)
    pad_bat = G + (jnp.arange(NPOOL - N, dtype=jnp.int32) % 16)
    bat_all = jnp.concatenate([batch, pad_bat])
    bidx = bat_all.reshape(NS, PB, BSZ)
    zeros = jnp.zeros((RPS, HD), jnp.float32)

    deg_p, cnt_p = _sc_pre(dst_deg, bat_all)
    deg_t = deg_p.reshape(NW, NP_P)[:, :N].T
    cnt_t = cnt_p.reshape(NW, GP)[:, :G].T

    g1lo, g1hi, dinv = _tc_a(deg_t, x, W1)
    a1 = _sc_agg(g1lo, g1hi, src_s, dst_s, zeros)
    g2lo, g2hi = _tc_b(a1, g1lo, g1hi, dinv, b1.reshape(1, H), W2)
    a2 = _sc_agg(g2lo, g2hi, src_s, dst_s, zeros)
    c2lo, c2hi = _tc_c(a2, g2lo, g2hi, dinv, b2.reshape(1, H))
    pool_p = _sc_pool(c2lo, c2hi, bidx, zeros)
    out = _tc_d(pool_p, cnt_t, Wf1, bf1.reshape(1, 128), Wf2,
                bf2.reshape(1, 1))
    return out


# speedup vs baseline: 18.5061x; 1.0168x over previous
import functools

import jax
import jax.numpy as jnp
from jax import lax
from jax.experimental import pallas as pl
from jax.experimental.pallas import tpu as pltpu
from jax.experimental.pallas import tpu_sc as plsc

NC, NS, L = 2, 16, 16
NW = NC * NS

N, E, D, H, G = 10000, 160000, 256, 256, 64
HD = D // 2

EPW = E // NW
EPW_P = 5008
EPS = E // NS
BSZ = 128
NB = 80
NBH = NB // 2
EPS_P = NB * BSZ
PADE = EPS_P - EPS
NP_P = 10112
RPS = NP_P // NS
NPOOL = 10240
BPW = NPOOL // NW
GP = 80

_mesh = plsc.VectorSubcoreMesh(core_axis_name="c", subcore_axis_name="s",
                               num_cores=NC, num_subcores=NS)
_sc_params = pltpu.CompilerParams(needs_layout_passes=False)


def _zero_vmem(ref, n):
    def body(i, _):
        ref[pl.ds(i * L, L)] = jnp.zeros((L,), jnp.float32)
        return 0
    lax.fori_loop(0, n // L, body, 0)


@functools.partial(
    pl.kernel,
    out_type=[jax.ShapeDtypeStruct((NW * NP_P,), jnp.float32),
              jax.ShapeDtypeStruct((NW * GP,), jnp.float32)],
    mesh=_mesh,
    compiler_params=_sc_params,
    scratch_types=[pltpu.VMEM((EPW_P,), jnp.int32),
                   pltpu.VMEM((BPW,), jnp.int32),
                   pltpu.VMEM((NP_P,), jnp.float32),
                   pltpu.VMEM((GP,), jnp.float32)],
)
def _sc_pre(dst_hbm, bat_hbm, deg_out, cnt_out, dstv, batv, dacc, cacc):
    c = lax.axis_index("c")
    s = lax.axis_index("s")
    wid = s * NC + c
    pltpu.sync_copy(dst_hbm.at[pl.ds(wid * EPW_P, EPW_P)], dstv)
    pltpu.sync_copy(bat_hbm.at[pl.ds(wid * BPW, BPW)], batv)
    _zero_vmem(dacc, NP_P)
    _zero_vmem(cacc, GP)
    ones = jnp.ones((L,), jnp.float32)

    def deg_body(i, _):
        d = dstv[pl.ds(i * L, L)]
        idx = jnp.where(d >= 0, d, N)
        plsc.addupdate_scatter(dacc, [idx], ones)
        return 0
    lax.fori_loop(0, EPW_P // L, deg_body, 0)

    def cnt_body(i, _):
        b = batv[pl.ds(i * L, L)]
        plsc.addupdate_scatter(cacc, [b], ones)
        return 0
    lax.fori_loop(0, BPW // L, cnt_body, 0)

    pltpu.sync_copy(dacc, deg_out.at[pl.ds(wid * NP_P, NP_P)])
    pltpu.sync_copy(cacc, cnt_out.at[pl.ds(wid * GP, GP)])


@functools.partial(
    pl.kernel,
    out_type=jax.ShapeDtypeStruct((NC, NP_P, HD), jnp.float32),
    mesh=_mesh,
    compiler_params=_sc_params,
    scratch_types=[pltpu.VMEM((NBH, BSZ), jnp.int32),
                   pltpu.VMEM((NBH, BSZ), jnp.int32),
                   pltpu.VMEM((BSZ, HD), jnp.float32),
                   pltpu.VMEM((BSZ, HD), jnp.float32),
                   pltpu.VMEM_SHARED((NP_P, HD), jnp.float32),
                   pltpu.SemaphoreType.DMA,
                   pltpu.SemaphoreType.DMA],
)
def _sc_agg(glo_hbm, ghi_hbm, src_hbm, dst_hbm, zeros_hbm, out_hbm,
            srcv, dstv, rbuf0, rbuf1, acc, sem0, sem1):
    c = lax.axis_index("c")
    s = lax.axis_index("s")
    pltpu.sync_copy(zeros_hbm.at[pl.ds(0, RPS)], acc.at[pl.ds(s * RPS, RPS)])
    plsc.subcore_barrier()

    def run(gref):
        def phase(p):
            pltpu.sync_copy(src_hbm.at[s, pl.ds(p * NBH, NBH)], srcv)
            pltpu.sync_copy(dst_hbm.at[s, pl.ds(p * NBH, NBH)], dstv)
            pltpu.async_copy(gref.at[srcv.at[0]], rbuf0, sem0)
            pltpu.async_copy(gref.at[srcv.at[1]], rbuf1, sem1)

            def step(b, rbuf, sem, refill):
                pltpu.make_async_copy(gref.at[srcv.at[b]], rbuf, sem).wait()
                pltpu.sync_copy(rbuf, acc.at[dstv.at[b]], add=True)
                if refill:
                    pltpu.async_copy(gref.at[srcv.at[b + 2]], rbuf, sem)

            def body(g, _):
                step(2 * g, rbuf0, sem0, True)
                step(2 * g + 1, rbuf1, sem1, True)
                return 0
            lax.fori_loop(0, (NBH - 2) // 2, body, 0)
            step(NBH - 2, rbuf0, sem0, False)
            step(NBH - 1, rbuf1, sem1, False)

        phase(0)
        phase(1)

    @pl.when(c == 0)
    def _():
        run(glo_hbm)

    @pl.when(c == 1)
    def _():
        run(ghi_hbm)

    plsc.subcore_barrier()
    pltpu.sync_copy(acc.at[pl.ds(s * RPS, RPS)],
                    out_hbm.at[c, pl.ds(s * RPS, RPS)])


R = 1000


def _tc_a_body(degp, x, w1, glo, ghi, dinv):
    deg = 1.0 + jnp.sum(degp[...], axis=1)
    di = lax.rsqrt(deg)[:, None]
    z = jnp.dot(x[...], w1[...], preferred_element_type=jnp.float32)
    g = z * di
    glo[...] = g[:, :HD]
    ghi[...] = g[:, HD:]
    dinv[...] = di


def _tc_a(deg_t, x, w1):
    return pl.pallas_call(
        _tc_a_body,
        grid=(N // R,),
        in_specs=[pl.BlockSpec((R, NW), lambda i: (i, 0)),
                  pl.BlockSpec((R, D), lambda i: (i, 0)),
                  pl.BlockSpec((D, H), lambda i: (0, 0))],
        out_specs=[pl.BlockSpec((R, HD), lambda i: (i, 0)),
                   pl.BlockSpec((R, HD), lambda i: (i, 0)),
                   pl.BlockSpec((R, 1), lambda i: (i, 0))],
        out_shape=[jax.ShapeDtypeStruct((N, HD), jnp.float32),
                   jax.ShapeDtypeStruct((N, HD), jnp.float32),
                   jax.ShapeDtypeStruct((N, 1), jnp.float32)],
    )(deg_t, x, w1)


def _tc_b_body(alo, ahi, glo, ghi, dinv, b1, w2, g2lo, g2hi):
    agg = jnp.concatenate([alo[0], ahi[0]], axis=1)
    g = jnp.concatenate([glo[...], ghi[...]], axis=1)
    h1 = jax.nn.relu(dinv[...] * (agg + g) + b1[...])
    z2 = jnp.dot(h1, w2[...], preferred_element_type=jnp.float32)
    g2 = z2 * dinv[...]
    g2lo[...] = g2[:, :HD]
    g2hi[...] = g2[:, HD:]


def _tc_b(a1, glo, ghi, dinv, b1, w2):
    return pl.pallas_call(
        _tc_b_body,
        grid=(N // R,),
        in_specs=[pl.BlockSpec((1, R, HD), lambda i: (0, i, 0)),
                  pl.BlockSpec((1, R, HD), lambda i: (1, i, 0)),
                  pl.BlockSpec((R, HD), lambda i: (i, 0)),
                  pl.BlockSpec((R, HD), lambda i: (i, 0)),
                  pl.BlockSpec((R, 1), lambda i: (i, 0)),
                  pl.BlockSpec((1, H), lambda i: (0, 0)),
                  pl.BlockSpec((H, H), lambda i: (0, 0))],
        out_specs=[pl.BlockSpec((R, HD), lambda i: (i, 0)),
                   pl.BlockSpec((R, HD), lambda i: (i, 0))],
        out_shape=[jax.ShapeDtypeStruct((N, HD), jnp.float32),
                   jax.ShapeDtypeStruct((N, HD), jnp.float32)],
    )(a1, a1, glo, ghi, dinv, b1, w2)


def _tc_cd_body(alo, ahi, glo, ghi, dinv, bat, b2, cntp, wf1, bf1, wf2, bf2,
                out, pacc):
    i = pl.program_id(0)
    agg = jnp.concatenate([alo[0], ahi[0]], axis=1)
    g = jnp.concatenate([glo[...], ghi[...]], axis=1)
    conv = dinv[...] * (agg + g) + b2[...]
    gid = lax.broadcasted_iota(jnp.int32, (1, G), 1)
    p = (bat[...] == gid).astype(jnp.float32)
    part = lax.dot_general(p, conv, (((0,), (0,)), ((), ())),
                           preferred_element_type=jnp.float32)

    @pl.when(i == 0)
    def _():
        pacc[...] = part

    @pl.when(i > 0)
    def _():
        pacc[...] = pacc[...] + part

    @pl.when(i == N // R - 1)
    def _():
        cnt = jnp.sum(cntp[...], axis=1)
        pooled = pacc[...] / jnp.maximum(cnt, 1.0)[:, None]
        s1 = jax.nn.relu(
            jnp.dot(pooled, wf1[...], preferred_element_type=jnp.float32)
            + bf1[...])
        out[...] = jnp.dot(s1, wf2[...],
                           preferred_element_type=jnp.float32) + bf2[...]


def _tc_cd(a2, g2lo, g2hi, dinv, bat_row, b2, cnt_t, wf1, bf1, wf2, bf2):
    return pl.pallas_call(
        _tc_cd_body,
        grid=(N // R,),
        in_specs=[pl.BlockSpec((1, R, HD), lambda i: (0, i, 0)),
                  pl.BlockSpec((1, R, HD), lambda i: (1, i, 0)),
                  pl.BlockSpec((R, HD), lambda i: (i, 0)),
                  pl.BlockSpec((R, HD), lambda i: (i, 0)),
                  pl.BlockSpec((R, 1), lambda i: (i, 0)),
                  pl.BlockSpec((R, 1), lambda i: (i, 0)),
                  pl.BlockSpec((1, H), lambda i: (0, 0)),
                  pl.BlockSpec((G, NW), lambda i: (0, 0)),
                  pl.BlockSpec((H, 128), lambda i: (0, 0)),
                  pl.BlockSpec((1, 128), lambda i: (0, 0)),
                  pl.BlockSpec((128, 1), lambda i: (0, 0)),
                  pl.BlockSpec((1, 1), lambda i: (0, 0))],
        out_specs=pl.BlockSpec((G, 1), lambda i: (0, 0)),
        out_shape=jax.ShapeDtypeStruct((G, 1), jnp.float32),
        scratch_shapes=[pltpu.VMEM((G, H), jnp.float32)],
    )(a2, a2, g2lo, g2hi, dinv, bat_row, b2, cnt_t, wf1, bf1, wf2, bf2)


@jax.jit
def kernel(x, edge_index, batch, W1, b1, W2, b2, Wf1, bf1, Wf2, bf2):
    src, dst = edge_index[0], edge_index[1]

    spread = jnp.arange(PADE, dtype=jnp.int32) % 16
    pad_blk = jnp.broadcast_to(spread, (NS, PADE))
    src_s = jnp.concatenate([src.reshape(NS, EPS), pad_blk],
                            axis=1).reshape(NS, NB, BSZ)
    dst_s = jnp.concatenate([dst.reshape(NS, EPS), N + pad_blk],
                            axis=1).reshape(NS, NB, BSZ)
    dst_deg = jnp.pad(dst.reshape(NW, EPW), ((0, 0), (0, EPW_P - EPW)),
                      constant_values=-1).reshape(-1)
    pad_bat = G + (jnp.arange(NPOOL - N, dtype=jnp.int32) % 16)
    bat_all = jnp.concatenate([batch, pad_bat])
    zeros = jnp.zeros((RPS, HD), jnp.float32)

    deg_p, cnt_p = _sc_pre(dst_deg, bat_all)
    deg_t = deg_p.reshape(NW, NP_P)[:, :N].T
    cnt_t = cnt_p.reshape(NW, GP)[:, :G].T

    g1lo, g1hi, dinv = _tc_a(deg_t, x, W1)
    a1 = _sc_agg(g1lo, g1hi, src_s, dst_s, zeros)
    g2lo, g2hi = _tc_b(a1, g1lo, g1hi, dinv, b1.reshape(1, H), W2)
    a2 = _sc_agg(g2lo, g2hi, src_s, dst_s, zeros)
    out = _tc_cd(a2, g2lo, g2hi, dinv, batch.reshape(N, 1), b2.reshape(1, H),
                 cnt_t, Wf1, bf1.reshape(1, 128), Wf2, bf2.reshape(1, 1))
    return out
